# Initial kernel scaffold; baseline (speedup 1.0000x reference)
import jax
import jax.numpy as jnp
from jax.experimental import pallas as pl


def kernel(x, weight):
    raise NotImplementedError("write your pallas kernel here")


# speedup vs baseline: 1.8225x; 1.8225x over previous
import jax
import jax.numpy as jnp
from jax.experimental import pallas as pl
from jax.experimental.pallas import tpu as pltpu

_K = 3


def _morph_body(w_ref, x_ref, o_ref, dscr, escr, xscr):
    H = x_ref.shape[1]
    W = x_ref.shape[2]
    out_h, out_w = H - _K + 1, W - _K + 1
    Rh, Rw = H - _K - 1, W - _K - 1
    f32 = x_ref.dtype

    row = jax.lax.broadcasted_iota(jnp.int32, (out_h, out_w), 0)
    col = jax.lax.broadcasted_iota(jnp.int32, (out_h, out_w), 1)
    interior = (row < Rh) & (col < Rw)

    xscr[...] = x_ref[0]

    pos_inf = jnp.array(jnp.inf, f32)
    dil = jnp.full((out_h, out_w), -pos_inf, f32)
    ero = jnp.full((out_h, out_w), pos_inf, f32)
    for i in range(_K):
        for j in range(_K):
            wij = w_ref[0, 0, _K * i + j]
            xs = xscr[i:i + out_h, j:j + out_w]
            dil = jnp.maximum(dil, jnp.abs(xs * wij))
            ero = jnp.minimum(ero, jnp.abs(xs + wij))
    dil = jnp.where(interior, dil, 0.0)
    ero = jnp.where(interior, ero, 0.0)

    dscr[...] = jnp.zeros((H, W), f32)
    escr[...] = jnp.zeros((H, W), f32)
    dscr[0:out_h, 0:out_w] = dil
    escr[0:out_h, 0:out_w] = ero

    opening = jnp.full((out_h, out_w), -pos_inf, f32)
    closing = jnp.full((out_h, out_w), pos_inf, f32)
    for i in range(_K):
        for j in range(_K):
            wij = w_ref[0, 0, _K * i + j]
            es = escr[i:i + out_h, j:j + out_w]
            ds = dscr[i:i + out_h, j:j + out_w]
            opening = jnp.maximum(opening, jnp.abs(es * wij))
            closing = jnp.minimum(closing, jnp.abs(ds + wij))
    opening = jnp.where(interior, opening, 0.0)
    closing = jnp.where(interior, closing, 0.0)

    o_ref[0, 0] = dil
    o_ref[0, 1] = ero
    o_ref[0, 2] = opening
    o_ref[0, 3] = closing


def _build(B, G, H, W, dtype, interpret=False):
    out_h, out_w = H - _K + 1, W - _K + 1
    return pl.pallas_call(
        _morph_body,
        out_shape=jax.ShapeDtypeStruct((B, 4 * G, out_h, out_w), dtype),
        grid=(B, G),
        in_specs=[
            pl.BlockSpec((1, 1, _K * _K), lambda b, g: (g, 0, 0)),
            pl.BlockSpec((1, H, W), lambda b, g: (b, 0, 0)),
        ],
        out_specs=pl.BlockSpec((1, 4, out_h, out_w), lambda b, g: (b, g, 0, 0)),
        scratch_shapes=[
            pltpu.VMEM((H, W), dtype),
            pltpu.VMEM((H, W), dtype),
            pltpu.VMEM((H, W), dtype),
        ],
        compiler_params=pltpu.CompilerParams(
            dimension_semantics=("parallel", "arbitrary"),
            vmem_limit_bytes=56 * 1024 * 1024,
        ),
        name="morph2d",
        interpret=interpret,
    )


def kernel(x, weight):
    B, _, H, W = x.shape
    G = weight.shape[0]
    x2 = x[:, 0]
    w2 = weight[:, 0].reshape(G, 1, _K * _K)
    return _build(B, G, H, W, x.dtype)(w2, x2)


# speedup vs baseline: 2.0684x; 1.1349x over previous
import jax
import jax.numpy as jnp
from jax.experimental import pallas as pl
from jax.experimental.pallas import tpu as pltpu

_K = 3
_STRIP = 64


def _morph_body(w_ref, x_ref, o_ref, dscr, escr, xscr, ascr):
    H = x_ref.shape[1]
    W = x_ref.shape[2]
    out_h, out_w = H - _K + 1, W - _K + 1
    Rh, Rw = H - _K - 1, W - _K - 1
    f32 = x_ref.dtype

    w = [w_ref[0, 0, k] for k in range(_K * _K)]

    xv = x_ref[0]
    xscr[...] = xv
    ascr[...] = jnp.abs(xv)

    mc = max(0, W - _STRIP)
    mr = max(0, H - _STRIP)
    zmargin = jnp.zeros((H, W - mc), f32)
    dscr[:, mc:] = zmargin
    escr[:, mc:] = zmargin
    zrows = jnp.zeros((H - mr, W), f32)
    dscr[mr:, :] = zrows
    escr[mr:, :] = zrows

    col = jax.lax.broadcasted_iota(jnp.int32, (1, out_w), 1)

    for r0 in range(0, out_h, _STRIP):
        S = min(_STRIP, out_h - r0)
        dil = jnp.zeros((S, out_w), f32)
        ero = jnp.full((S, out_w), jnp.inf, f32)
        for i in range(_K):
            for j in range(_K):
                wij = w[_K * i + j]
                xs = xscr[r0 + i:r0 + i + S, j:j + out_w]
                ab = ascr[r0 + i:r0 + i + S, j:j + out_w]
                dil = jnp.maximum(dil, ab * wij)
                ero = jnp.minimum(ero, jnp.abs(xs + wij))
        row = r0 + jax.lax.broadcasted_iota(jnp.int32, (S, 1), 0)
        interior = (row < Rh) & (col < Rw)
        dil = jnp.where(interior, dil, 0.0)
        ero = jnp.where(interior, ero, 0.0)
        dscr[r0:r0 + S, 0:out_w] = dil
        escr[r0:r0 + S, 0:out_w] = ero
        o_ref[0, 0, r0:r0 + S, :] = dil
        o_ref[0, 1, r0:r0 + S, :] = ero

    for r0 in range(0, out_h, _STRIP):
        S = min(_STRIP, out_h - r0)
        opening = jnp.zeros((S, out_w), f32)
        closing = jnp.full((S, out_w), jnp.inf, f32)
        for i in range(_K):
            for j in range(_K):
                wij = w[_K * i + j]
                es = escr[r0 + i:r0 + i + S, j:j + out_w]
                ds = dscr[r0 + i:r0 + i + S, j:j + out_w]
                opening = jnp.maximum(opening, es * wij)
                closing = jnp.minimum(closing, ds + wij)
        row = r0 + jax.lax.broadcasted_iota(jnp.int32, (S, 1), 0)
        interior = (row < Rh) & (col < Rw)
        closing = jnp.where(interior, closing, 0.0)
        o_ref[0, 2, r0:r0 + S, :] = opening
        o_ref[0, 3, r0:r0 + S, :] = closing


def _build(B, G, H, W, dtype, interpret=False):
    out_h, out_w = H - _K + 1, W - _K + 1
    return pl.pallas_call(
        _morph_body,
        out_shape=jax.ShapeDtypeStruct((B, 4 * G, out_h, out_w), dtype),
        grid=(B, G),
        in_specs=[
            pl.BlockSpec((1, 1, _K * _K), lambda b, g: (g, 0, 0)),
            pl.BlockSpec((1, H, W), lambda b, g: (b, 0, 0)),
        ],
        out_specs=pl.BlockSpec((1, 4, out_h, out_w), lambda b, g: (b, g, 0, 0)),
        scratch_shapes=[
            pltpu.VMEM((H, W), dtype),
            pltpu.VMEM((H, W), dtype),
            pltpu.VMEM((H, W), dtype),
            pltpu.VMEM((H, W), dtype),
        ],
        compiler_params=pltpu.CompilerParams(
            dimension_semantics=("parallel", "arbitrary"),
            vmem_limit_bytes=56 * 1024 * 1024,
        ),
        name="morph2d",
        interpret=interpret,
    )


def kernel(x, weight):
    B, _, H, W = x.shape
    G = weight.shape[0]
    x2 = x[:, 0]
    w2 = weight[:, 0].reshape(G, 1, _K * _K)
    return _build(B, G, H, W, x.dtype)(w2, x2)
